# Initial kernel scaffold; baseline (speedup 1.0000x reference)
import functools
import numpy as np
import jax
import jax.numpy as jnp
from jax import lax
from jax.experimental import pallas as pl
from jax.experimental.pallas import tpu as pltpu
from jax.experimental.pallas import tpu_sc as plsc

N = 10000
E = 320000
B = 16
NPG = N // B
D_IN = 128
D_OUT = 300

NC = 2
NS = 16
NW = NC * NS
E_PER_W = E // NW
CHUNK = 80
NCHUNK = E_PER_W // CHUNK
ROWS_PER_TILE = N // NS
ZROWS = 125
LANES = 16
KSUB = D_IN // LANES


def _sc_aggregate_body(x_hbm, src_hbm, dst_hbm, w_hbm, out_hbm,
                       agg_sh, idx_s, idx_d, w_v, rows_v, zbuf, sem):
    cid = lax.axis_index("c")
    sid = lax.axis_index("s")
    wid = sid * NC + cid

    def zero_row(r, _):
        for k in range(KSUB):
            zbuf[r, pl.ds(k * LANES, LANES)] = jnp.zeros((LANES,), jnp.float32)
        return 0
    lax.fori_loop(0, ZROWS, zero_row, 0)
    row0 = pl.multiple_of(sid * ROWS_PER_TILE, 8)
    for j in range(ROWS_PER_TILE // ZROWS):
        pltpu.sync_copy(zbuf, agg_sh.at[pl.ds(row0 + j * ZROWS, ZROWS)])
    plsc.subcore_barrier()

    e0 = wid * E_PER_W

    def chunk_body(i, _):
        base = pl.multiple_of(e0 + i * CHUNK, 8)
        pltpu.sync_copy(src_hbm.at[pl.ds(base, CHUNK)], idx_s)
        pltpu.sync_copy(dst_hbm.at[pl.ds(base, CHUNK)], idx_d)
        pltpu.sync_copy(w_hbm.at[pl.ds(base, CHUNK)], w_v)
        pltpu.async_copy(x_hbm.at[idx_s], rows_v, sem).wait()

        def edge_body(e, _):
            w = w_v[e]
            for k in range(KSUB):
                sl = pl.ds(k * LANES, LANES)
                rows_v[e, sl] = rows_v[e, sl] * w
            return 0
        lax.fori_loop(0, CHUNK, edge_body, 0)

        pltpu.sync_copy(rows_v, agg_sh.at[idx_d], add=True)
        return 0
    lax.fori_loop(0, NCHUNK, chunk_body, 0)
    plsc.subcore_barrier()

    pltpu.sync_copy(agg_sh.at[pl.ds(row0, ROWS_PER_TILE)],
                    out_hbm.at[cid, pl.ds(row0, ROWS_PER_TILE)])


_sc_aggregate = pl.kernel(
    _sc_aggregate_body,
    out_type=jax.ShapeDtypeStruct((NC, N, D_IN), jnp.float32),
    mesh=plsc.VectorSubcoreMesh(core_axis_name="c", subcore_axis_name="s"),
    scratch_types=[
        pltpu.VMEM_SHARED((N, D_IN), jnp.float32),
        pltpu.VMEM((CHUNK,), jnp.int32),
        pltpu.VMEM((CHUNK,), jnp.int32),
        pltpu.VMEM((CHUNK,), jnp.float32),
        pltpu.VMEM((CHUNK, D_IN), jnp.float32),
        pltpu.VMEM((ZROWS, D_IN), jnp.float32),
        pltpu.SemaphoreType.DMA,
    ],
)


ROWS_PER_STEP = 400
NSTEPS = N // ROWS_PER_STEP


def _tc_head_body(agg_ref, w_ref, b_ref, a_ref, m_ref,
                  pool_out, anc_out, acc_ref):
    i = pl.program_id(0)

    @pl.when(i == 0)
    def _init():
        acc_ref[...] = jnp.zeros_like(acc_ref)

    agg = agg_ref[0] + agg_ref[1]
    h = jnp.dot(agg, w_ref[...], preferred_element_type=jnp.float32)
    h = h + b_ref[...]
    a = a_ref[0, 0]
    h = jnp.where(h >= 0.0, h, a * h)
    acc_ref[...] += jnp.dot(m_ref[...], h, preferred_element_type=jnp.float32)

    @pl.when(i == NSTEPS - 1)
    def _finish():
        pooled = acc_ref[...]
        nrm = jnp.sqrt(jnp.sum(pooled * pooled, axis=1, keepdims=True))
        pooled = pooled / jnp.maximum(nrm, 1e-12)
        pool_out[...] = pooled[:B, :]
        anc_out[...] = pooled[B:, :]


_tc_head = pl.pallas_call(
    _tc_head_body,
    grid=(NSTEPS,),
    in_specs=[
        pl.BlockSpec((NC, ROWS_PER_STEP, D_IN), lambda i: (0, i, 0)),
        pl.BlockSpec((D_IN, D_OUT), lambda i: (0, 0)),
        pl.BlockSpec((1, D_OUT), lambda i: (0, 0)),
        pl.BlockSpec((1, 1), lambda i: (0, 0)),
        pl.BlockSpec((2 * B, ROWS_PER_STEP), lambda i: (0, i)),
    ],
    out_specs=[
        pl.BlockSpec((B, D_OUT), lambda i: (0, 0)),
        pl.BlockSpec((B, D_OUT), lambda i: (0, 0)),
    ],
    out_shape=[
        jax.ShapeDtypeStruct((B, D_OUT), jnp.float32),
        jax.ShapeDtypeStruct((B, D_OUT), jnp.float32),
    ],
    scratch_shapes=[pltpu.VMEM((2 * B, D_OUT), jnp.float32)],
)


def _pool_matrix():
    m = np.zeros((2 * B, N), dtype=np.float32)
    for g in range(B):
        m[g, g * NPG:(g + 1) * NPG - 1] = 1.0 / (NPG - 1)
        m[B + g, (g + 1) * NPG - 1] = 1.0
    return jnp.asarray(m)


_POOL_M = _pool_matrix()


def kernel(in_feat, edge_weight, W, b, prelu_a, edge_index):
    src = edge_index[0]
    dst = edge_index[1]
    agg = _sc_aggregate(in_feat, src, dst, edge_weight)
    pool, anchor = _tc_head(
        agg, W,
        b.reshape(1, D_OUT),
        prelu_a.reshape(1, 1),
        _POOL_M,
    )
    return (pool, anchor)


# speedup vs baseline: 5.8269x; 5.8269x over previous
import functools
import numpy as np
import jax
import jax.numpy as jnp
from jax import lax
from jax.experimental import pallas as pl
from jax.experimental.pallas import tpu as pltpu
from jax.experimental.pallas import tpu_sc as plsc

N = 10000
N_PAD = 10240
E = 320000
B = 16
NPG = N // B
D_IN = 128
D_OUT = 300

NC = 2
NS = 16
NW = NC * NS
E_PER_W = E // NW
CHUNK = 80
NCHUNK = E_PER_W // CHUNK
ROWS_PER_TILE = N_PAD // NS
ZROWS = 128
LANES = 16
KSUB = D_IN // LANES


def _sc_aggregate_body(x_hbm, src_hbm, dst_hbm, w_hbm, out_hbm,
                       agg_sh, idx_s, idx_d, w_v, rows_v, zbuf, sem):
    cid = lax.axis_index("c")
    sid = lax.axis_index("s")
    wid = sid * NC + cid

    def zero_row(r, _):
        for k in range(KSUB):
            zbuf[r, pl.ds(k * LANES, LANES)] = jnp.zeros((LANES,), jnp.float32)
        return 0
    lax.fori_loop(0, ZROWS, zero_row, 0)
    row0 = pl.multiple_of(sid * ROWS_PER_TILE, 8)
    for j in range(ROWS_PER_TILE // ZROWS):
        pltpu.sync_copy(zbuf, agg_sh.at[pl.ds(row0 + j * ZROWS, ZROWS)])
    plsc.subcore_barrier()

    e0 = wid * E_PER_W

    def chunk_body(i, _):
        base = pl.multiple_of(e0 + i * CHUNK, 8)
        pltpu.sync_copy(src_hbm.at[pl.ds(base, CHUNK)], idx_s)
        pltpu.sync_copy(dst_hbm.at[pl.ds(base, CHUNK)], idx_d)
        pltpu.sync_copy(w_hbm.at[pl.ds(base, CHUNK)], w_v)
        pltpu.async_copy(x_hbm.at[idx_s], rows_v, sem).wait()

        def edge_body(e, _):
            w = plsc.load_gather(w_v, [jnp.full((LANES,), e, jnp.int32)])
            for k in range(KSUB):
                sl = pl.ds(k * LANES, LANES)
                rows_v[e, sl] = rows_v[e, sl] * w
            return 0
        lax.fori_loop(0, CHUNK, edge_body, 0)

        pltpu.sync_copy(rows_v, agg_sh.at[idx_d], add=True)
        return 0
    lax.fori_loop(0, NCHUNK, chunk_body, 0)
    plsc.subcore_barrier()

    pltpu.sync_copy(agg_sh.at[pl.ds(row0, ROWS_PER_TILE)],
                    out_hbm.at[cid, pl.ds(row0, ROWS_PER_TILE)])


@functools.cache
def _sc_aggregate():
    return pl.kernel(
        _sc_aggregate_body,
        out_type=jax.ShapeDtypeStruct((NC, N_PAD, D_IN), jnp.float32),
        mesh=plsc.VectorSubcoreMesh(core_axis_name="c", subcore_axis_name="s",
                                    num_cores=NC, num_subcores=NS),
        compiler_params=pltpu.CompilerParams(needs_layout_passes=False),
        scratch_types=[
            pltpu.VMEM_SHARED((N_PAD, D_IN), jnp.float32),
            pltpu.VMEM((CHUNK,), jnp.int32),
            pltpu.VMEM((CHUNK,), jnp.int32),
            pltpu.VMEM((CHUNK,), jnp.float32),
            pltpu.VMEM((CHUNK, D_IN), jnp.float32),
            pltpu.VMEM((ZROWS, D_IN), jnp.float32),
            pltpu.SemaphoreType.DMA,
        ],
    )


ROWS_PER_STEP = 1280
NSTEPS = N_PAD // ROWS_PER_STEP


def _tc_head_body(agg_ref, w_ref, b_ref, a_ref, m_ref,
                  pool_out, anc_out, acc_ref):
    i = pl.program_id(0)

    @pl.when(i == 0)
    def _init():
        acc_ref[...] = jnp.zeros_like(acc_ref)

    agg = agg_ref[0] + agg_ref[1]
    h = jnp.dot(agg, w_ref[...], preferred_element_type=jnp.float32)
    h = h + b_ref[...]
    a = a_ref[0, 0]
    h = jnp.where(h >= 0.0, h, a * h)
    acc_ref[...] += jnp.dot(m_ref[...], h, preferred_element_type=jnp.float32)

    @pl.when(i == NSTEPS - 1)
    def _finish():
        pooled = acc_ref[...]
        nrm = jnp.sqrt(jnp.sum(pooled * pooled, axis=1, keepdims=True))
        pooled = pooled / jnp.maximum(nrm, 1e-12)
        pool_out[...] = pooled[:B, :]
        anc_out[...] = pooled[B:, :]


_tc_head = pl.pallas_call(
    _tc_head_body,
    grid=(NSTEPS,),
    in_specs=[
        pl.BlockSpec((NC, ROWS_PER_STEP, D_IN), lambda i: (0, i, 0)),
        pl.BlockSpec((D_IN, D_OUT), lambda i: (0, 0)),
        pl.BlockSpec((1, D_OUT), lambda i: (0, 0)),
        pl.BlockSpec((1, 1), lambda i: (0, 0)),
        pl.BlockSpec((2 * B, ROWS_PER_STEP), lambda i: (0, i)),
    ],
    out_specs=[
        pl.BlockSpec((B, D_OUT), lambda i: (0, 0)),
        pl.BlockSpec((B, D_OUT), lambda i: (0, 0)),
    ],
    out_shape=[
        jax.ShapeDtypeStruct((B, D_OUT), jnp.float32),
        jax.ShapeDtypeStruct((B, D_OUT), jnp.float32),
    ],
    scratch_shapes=[pltpu.VMEM((2 * B, D_OUT), jnp.float32)],
)


def _pool_matrix():
    m = np.zeros((2 * B, N_PAD), dtype=np.float32)
    for g in range(B):
        m[g, g * NPG:(g + 1) * NPG - 1] = 1.0 / (NPG - 1)
        m[B + g, (g + 1) * NPG - 1] = 1.0
    return m


_POOL_M = _pool_matrix()


def kernel(in_feat, edge_weight, W, b, prelu_a, edge_index):
    src = edge_index[0]
    dst = edge_index[1]
    agg = _sc_aggregate()(in_feat, src, dst, edge_weight)
    pool, anchor = _tc_head(
        agg, W,
        b.reshape(1, D_OUT),
        prelu_a.reshape(1, 1),
        jnp.asarray(_POOL_M),
    )
    return (pool, anchor)


# speedup vs baseline: 6.1135x; 1.0492x over previous
import functools
import numpy as np
import jax
import jax.numpy as jnp
from jax import lax
from jax.experimental import pallas as pl
from jax.experimental.pallas import tpu as pltpu
from jax.experimental.pallas import tpu_sc as plsc

N = 10000
N_PAD = 10240
E = 320000
B = 16
NPG = N // B
D_IN = 128
D_OUT = 300

NC = 2
NS = 16
NW = NC * NS
E_PAD = NW * 10240
E_PER_W = E_PAD // NW
IDXR = 80
CHUNK = 64
NCHUNK = E_PER_W // CHUNK
NIDX = 4
ROWS_PER_TILE = N_PAD // NS
ZROWS = 16
LANES = 16
KSUB = D_IN // LANES


NBUF = 2


def _sc_aggregate_body(x_hbm, src_hbm, dst_hbm, w_hbm, out_hbm,
                       agg_sh, src_t, dstr, wr, rows, zbuf,
                       ssrc, gsem, ssem, dsem, wsem):
    cid = lax.axis_index("c")
    sid = lax.axis_index("s")
    wid = sid * NC + cid

    csrc = pltpu.async_copy(src_hbm.at[wid], src_t, ssrc)

    def dst_desc(c, slot):
        return pltpu.make_async_copy(dst_hbm.at[wid, c], dstr.at[slot],
                                     dsem.at[slot])

    def w_desc(c, slot):
        return pltpu.make_async_copy(w_hbm.at[wid, c], wr.at[slot],
                                     wsem.at[slot])

    def prologue_idx(c, _):
        dst_desc(c, c).start()
        w_desc(c, c).start()
        return 0
    lax.fori_loop(0, 2, prologue_idx, 0)

    def zero_row(r, _):
        for k in range(KSUB):
            zbuf[r, pl.ds(k * LANES, LANES)] = jnp.zeros((LANES,),
                                                         jnp.float32)
        return 0
    lax.fori_loop(0, ZROWS, zero_row, 0)
    row0 = pl.multiple_of(sid * ROWS_PER_TILE, 8)

    def zero_copy(j, _):
        pltpu.sync_copy(zbuf, agg_sh.at[pl.ds(row0 + j * ZROWS, ZROWS)])
        return 0
    lax.fori_loop(0, ROWS_PER_TILE // ZROWS, zero_copy, 0)
    plsc.subcore_barrier()

    csrc.wait()

    def gather_desc(c, b):
        idx = src_t.at[lax.div(c, 2), pl.ds(lax.rem(c, 2) * CHUNK, CHUNK)]
        return pltpu.make_async_copy(x_hbm.at[idx], rows.at[b], gsem.at[b])

    def scatter_desc(slot, b):
        return pltpu.make_async_copy(rows.at[b], agg_sh.at[dstr.at[slot]],
                                     ssem.at[b])

    gather_desc(0, 0).start()

    def chunk_step(c, _):
        b = lax.rem(c, NBUF)
        ob = 1 - b
        slot = lax.rem(c, NIDX)

        gather_desc(c, b).wait()

        @pl.when(c >= 1)
        def _():
            scatter_desc(0, ob).wait()

        @pl.when(c + 1 < NCHUNK)
        def _():
            gather_desc(c + 1, ob).start()

        dst_desc(c, slot).wait()
        w_desc(c, slot).wait()

        def edge2(t, _):
            for u in range(2):
                e = t * 2 + u
                w = plsc.load_gather(
                    wr, [jnp.full((LANES,), slot, jnp.int32),
                         jnp.full((LANES,), e, jnp.int32)])
                for k in range(KSUB):
                    sl = pl.ds(k * LANES, LANES)
                    rows[b, e, sl] = rows[b, e, sl] * w
            return 0
        lax.fori_loop(0, CHUNK // 2, edge2, 0)

        @pl.when(c + 2 < NCHUNK)
        def _():
            slot2 = lax.rem(c + 2, NIDX)
            dst_desc(c + 2, slot2).start()
            w_desc(c + 2, slot2).start()

        pltpu.async_copy(rows.at[b], agg_sh.at[dstr.at[slot]], ssem.at[b],
                         add=True)
        return 0
    lax.fori_loop(0, NCHUNK, chunk_step, 0)

    scatter_desc(0, lax.rem(NCHUNK - 1, NBUF)).wait()
    plsc.subcore_barrier()

    pltpu.sync_copy(agg_sh.at[pl.ds(row0, ROWS_PER_TILE)],
                    out_hbm.at[cid, pl.ds(row0, ROWS_PER_TILE)])


@functools.cache
def _sc_aggregate():
    return pl.kernel(
        _sc_aggregate_body,
        out_type=jax.ShapeDtypeStruct((NC, N_PAD, D_IN), jnp.float32),
        mesh=plsc.VectorSubcoreMesh(core_axis_name="c", subcore_axis_name="s",
                                    num_cores=NC, num_subcores=NS),
        compiler_params=pltpu.CompilerParams(needs_layout_passes=False),
        scratch_types=(
            [pltpu.VMEM_SHARED((N_PAD, D_IN), jnp.float32),
             pltpu.VMEM((IDXR, 128), jnp.int32),
             pltpu.VMEM((NIDX, CHUNK), jnp.int32),
             pltpu.VMEM((NIDX, CHUNK), jnp.float32),
             pltpu.VMEM((NBUF, CHUNK, D_IN), jnp.float32),
             pltpu.VMEM((ZROWS, D_IN), jnp.float32),
             pltpu.SemaphoreType.DMA,
             pltpu.SemaphoreType.DMA((NBUF,)),
             pltpu.SemaphoreType.DMA((NBUF,)),
             pltpu.SemaphoreType.DMA((NIDX,)),
             pltpu.SemaphoreType.DMA((NIDX,))]
        ),
    )


ROWS_PER_STEP = 1280
NSTEPS = N_PAD // ROWS_PER_STEP


def _tc_head_body(agg_ref, w_ref, b_ref, a_ref, m_ref,
                  pool_out, anc_out, acc_ref):
    i = pl.program_id(0)

    @pl.when(i == 0)
    def _init():
        acc_ref[...] = jnp.zeros_like(acc_ref)

    agg = agg_ref[0] + agg_ref[1]
    h = jnp.dot(agg, w_ref[...], preferred_element_type=jnp.float32)
    h = h + b_ref[...]
    a = a_ref[0, 0]
    h = jnp.where(h >= 0.0, h, a * h)
    acc_ref[...] += jnp.dot(m_ref[...], h, preferred_element_type=jnp.float32)

    @pl.when(i == NSTEPS - 1)
    def _finish():
        pooled = acc_ref[...]
        nrm = jnp.sqrt(jnp.sum(pooled * pooled, axis=1, keepdims=True))
        pooled = pooled / jnp.maximum(nrm, 1e-12)
        pool_out[...] = pooled[:B, :]
        anc_out[...] = pooled[B:, :]


_tc_head = pl.pallas_call(
    _tc_head_body,
    grid=(NSTEPS,),
    in_specs=[
        pl.BlockSpec((NC, ROWS_PER_STEP, D_IN), lambda i: (0, i, 0)),
        pl.BlockSpec((D_IN, D_OUT), lambda i: (0, 0)),
        pl.BlockSpec((1, D_OUT), lambda i: (0, 0)),
        pl.BlockSpec((1, 1), lambda i: (0, 0)),
        pl.BlockSpec((2 * B, ROWS_PER_STEP), lambda i: (0, i)),
    ],
    out_specs=[
        pl.BlockSpec((B, D_OUT), lambda i: (0, 0)),
        pl.BlockSpec((B, D_OUT), lambda i: (0, 0)),
    ],
    out_shape=[
        jax.ShapeDtypeStruct((B, D_OUT), jnp.float32),
        jax.ShapeDtypeStruct((B, D_OUT), jnp.float32),
    ],
    scratch_shapes=[pltpu.VMEM((2 * B, D_OUT), jnp.float32)],
)


def _pool_matrix():
    m = np.zeros((2 * B, N_PAD), dtype=np.float32)
    for g in range(B):
        m[g, g * NPG:(g + 1) * NPG - 1] = 1.0 / (NPG - 1)
        m[B + g, (g + 1) * NPG - 1] = 1.0
    return m


_POOL_M = _pool_matrix()


def kernel(in_feat, edge_weight, W, b, prelu_a, edge_index):
    pad = E_PAD - E
    src = jnp.pad(edge_index[0], (0, pad)).reshape(NW, IDXR, 128)
    dst = jnp.pad(edge_index[1], (0, pad)).reshape(NW, NCHUNK, CHUNK)
    wgt = jnp.pad(edge_weight, (0, pad)).reshape(NW, NCHUNK, CHUNK)
    agg = _sc_aggregate()(in_feat, src, dst, wgt)
    pool, anchor = _tc_head(
        agg, W,
        b.reshape(1, D_OUT),
        prelu_a.reshape(1, 1),
        jnp.asarray(_POOL_M),
    )
    return (pool, anchor)


# speedup vs baseline: 6.6194x; 1.0827x over previous
import functools
import numpy as np
import jax
import jax.numpy as jnp
from jax import lax
from jax.experimental import pallas as pl
from jax.experimental.pallas import tpu as pltpu
from jax.experimental.pallas import tpu_sc as plsc

N = 10000
N_PAD = 10240
E = 320000
B = 16
NPG = N // B
D_IN = 128
D_OUT = 300

NC = 2
NS = 16
E_PAD = 327680
CHUNK = 64
E_CORE0 = 225280
E_CORE1 = E_PAD - E_CORE0
NCH0 = E_CORE0 // NS // CHUNK
NCH1 = E_CORE1 // NS // CHUNK
NIDX = 4
ROWS_PER_TILE = N_PAD // NS
ZROWS = 16
LANES = 16
KSUB = D_IN // LANES


NBUF = 2


def _sc_aggregate_body(x_hbm, src_hbm, dst_hbm, w_hbm, out_hbm,
                       agg_sh, srcr, dstr, wr, rows, zbuf,
                       gsem, ssem, srcsem, dsem, wsem):
    cid = lax.axis_index("c")
    sid = lax.axis_index("s")
    e0 = jnp.where(cid == 0, sid * (NCH0 * CHUNK),
                   E_CORE0 + sid * (NCH1 * CHUNK))
    nch = jnp.where(cid == 0, NCH0, NCH1)

    def src_desc(c, slot):
        return pltpu.make_async_copy(
            src_hbm.at[pl.ds(e0 + c * CHUNK, CHUNK)], srcr.at[slot],
            srcsem.at[slot])

    def dst_desc(c, slot):
        return pltpu.make_async_copy(
            dst_hbm.at[pl.ds(e0 + c * CHUNK, CHUNK)], dstr.at[slot],
            dsem.at[slot])

    def w_desc(c, slot):
        return pltpu.make_async_copy(
            w_hbm.at[pl.ds(e0 + c * CHUNK, CHUNK)], wr.at[slot],
            wsem.at[slot])

    def prologue_idx(c, _):
        src_desc(c, c).start()
        dst_desc(c, c).start()
        w_desc(c, c).start()
        return 0
    lax.fori_loop(0, 2, prologue_idx, 0)

    def zero_row(r, _):
        for k in range(KSUB):
            zbuf[r, pl.ds(k * LANES, LANES)] = jnp.zeros((LANES,),
                                                         jnp.float32)
        return 0
    lax.fori_loop(0, ZROWS, zero_row, 0)
    row0 = pl.multiple_of(sid * ROWS_PER_TILE, 8)

    def zero_copy(j, _):
        pltpu.sync_copy(zbuf, agg_sh.at[pl.ds(row0 + j * ZROWS, ZROWS)])
        return 0
    lax.fori_loop(0, ROWS_PER_TILE // ZROWS, zero_copy, 0)
    plsc.subcore_barrier()

    def gather_desc(slot, b):
        return pltpu.make_async_copy(x_hbm.at[srcr.at[slot]], rows.at[b],
                                     gsem.at[b])

    def scatter_desc(slot, b):
        return pltpu.make_async_copy(rows.at[b], agg_sh.at[dstr.at[slot]],
                                     ssem.at[b])

    src_desc(0, 0).wait()
    gather_desc(0, 0).start()

    def chunk_step(c, _):
        b = lax.rem(c, NBUF)
        ob = 1 - b
        slot = lax.rem(c, NIDX)

        gather_desc(slot, b).wait()

        @pl.when(c >= 1)
        def _():
            scatter_desc(0, ob).wait()

        @pl.when(c + 1 < nch)
        def _():
            slot1 = lax.rem(c + 1, NIDX)
            src_desc(0, slot1).wait()
            gather_desc(slot1, ob).start()

        dst_desc(0, slot).wait()
        w_desc(0, slot).wait()

        def edge2(t, _):
            for u in range(2):
                e = t * 2 + u
                w = plsc.load_gather(
                    wr, [jnp.full((LANES,), slot, jnp.int32),
                         jnp.full((LANES,), e, jnp.int32)])
                for k in range(KSUB):
                    sl = pl.ds(k * LANES, LANES)
                    rows[b, e, sl] = rows[b, e, sl] * w
            return 0
        lax.fori_loop(0, CHUNK // 2, edge2, 0)

        @pl.when(c + 2 < nch)
        def _():
            slot2 = lax.rem(c + 2, NIDX)
            src_desc(c + 2, slot2).start()
            dst_desc(c + 2, slot2).start()
            w_desc(c + 2, slot2).start()

        pltpu.async_copy(rows.at[b], agg_sh.at[dstr.at[slot]], ssem.at[b],
                         add=True)
        return 0
    lax.fori_loop(0, nch, chunk_step, 0)

    scatter_desc(0, lax.rem(nch - 1, NBUF)).wait()
    plsc.subcore_barrier()

    pltpu.sync_copy(agg_sh.at[pl.ds(row0, ROWS_PER_TILE)],
                    out_hbm.at[cid, pl.ds(row0, ROWS_PER_TILE)])


@functools.cache
def _sc_aggregate():
    return pl.kernel(
        _sc_aggregate_body,
        out_type=jax.ShapeDtypeStruct((NC, N_PAD, D_IN), jnp.float32),
        mesh=plsc.VectorSubcoreMesh(core_axis_name="c", subcore_axis_name="s",
                                    num_cores=NC, num_subcores=NS),
        compiler_params=pltpu.CompilerParams(needs_layout_passes=False),
        scratch_types=(
            [pltpu.VMEM_SHARED((N_PAD, D_IN), jnp.float32),
             pltpu.VMEM((NIDX, CHUNK), jnp.int32),
             pltpu.VMEM((NIDX, CHUNK), jnp.int32),
             pltpu.VMEM((NIDX, CHUNK), jnp.float32),
             pltpu.VMEM((NBUF, CHUNK, D_IN), jnp.float32),
             pltpu.VMEM((ZROWS, D_IN), jnp.float32),
             pltpu.SemaphoreType.DMA((NBUF,)),
             pltpu.SemaphoreType.DMA((NBUF,)),
             pltpu.SemaphoreType.DMA((NIDX,)),
             pltpu.SemaphoreType.DMA((NIDX,)),
             pltpu.SemaphoreType.DMA((NIDX,))]
        ),
    )


ROWS_PER_STEP = 1280
NSTEPS = N_PAD // ROWS_PER_STEP


def _tc_head_body(agg_ref, w_ref, b_ref, a_ref, m_ref,
                  pool_out, anc_out, acc_ref):
    i = pl.program_id(0)

    @pl.when(i == 0)
    def _init():
        acc_ref[...] = jnp.zeros_like(acc_ref)

    agg = agg_ref[0] + agg_ref[1]
    h = jnp.dot(agg, w_ref[...], preferred_element_type=jnp.float32)
    h = h + b_ref[...]
    a = a_ref[0, 0]
    h = jnp.where(h >= 0.0, h, a * h)
    acc_ref[...] += jnp.dot(m_ref[...], h, preferred_element_type=jnp.float32)

    @pl.when(i == NSTEPS - 1)
    def _finish():
        pooled = acc_ref[...]
        nrm = jnp.sqrt(jnp.sum(pooled * pooled, axis=1, keepdims=True))
        pooled = pooled / jnp.maximum(nrm, 1e-12)
        pool_out[...] = pooled[:B, :]
        anc_out[...] = pooled[B:, :]


_tc_head = pl.pallas_call(
    _tc_head_body,
    grid=(NSTEPS,),
    in_specs=[
        pl.BlockSpec((NC, ROWS_PER_STEP, D_IN), lambda i: (0, i, 0)),
        pl.BlockSpec((D_IN, D_OUT), lambda i: (0, 0)),
        pl.BlockSpec((1, D_OUT), lambda i: (0, 0)),
        pl.BlockSpec((1, 1), lambda i: (0, 0)),
        pl.BlockSpec((2 * B, ROWS_PER_STEP), lambda i: (0, i)),
    ],
    out_specs=[
        pl.BlockSpec((B, D_OUT), lambda i: (0, 0)),
        pl.BlockSpec((B, D_OUT), lambda i: (0, 0)),
    ],
    out_shape=[
        jax.ShapeDtypeStruct((B, D_OUT), jnp.float32),
        jax.ShapeDtypeStruct((B, D_OUT), jnp.float32),
    ],
    scratch_shapes=[pltpu.VMEM((2 * B, D_OUT), jnp.float32)],
)


def _pool_matrix():
    m = np.zeros((2 * B, N_PAD), dtype=np.float32)
    for g in range(B):
        m[g, g * NPG:(g + 1) * NPG - 1] = 1.0 / (NPG - 1)
        m[B + g, (g + 1) * NPG - 1] = 1.0
    return m


_POOL_M = _pool_matrix()


def kernel(in_feat, edge_weight, W, b, prelu_a, edge_index):
    pad = E_PAD - E
    src = jnp.pad(edge_index[0], (0, pad))
    dst = jnp.pad(edge_index[1], (0, pad))
    wgt = jnp.pad(edge_weight, (0, pad))
    agg = _sc_aggregate()(in_feat, src, dst, wgt)
    pool, anchor = _tc_head(
        agg, W,
        b.reshape(1, D_OUT),
        prelu_a.reshape(1, 1),
        jnp.asarray(_POOL_M),
    )
    return (pool, anchor)
